# Initial kernel scaffold; baseline (speedup 1.0000x reference)
import jax
import jax.numpy as jnp
from jax.experimental import pallas as pl


def kernel(x, edge_index_list, gcn_w, gcn_b, W1, W2, pred_w, pred_b):
    raise NotImplementedError("write your pallas kernel here")


# speedup vs baseline: 7.4169x; 7.4169x over previous
import functools

import jax
import jax.numpy as jnp
from jax import lax
from jax.experimental import pallas as pl
from jax.experimental.pallas import tpu as pltpu
from jax.experimental.pallas import tpu_sc as plsc

_N = 10000
_R = 3
_C = 128
_E = 320000
_NC = 2
_NS = 16
_NW = _NC * _NS
_PROWS = 2560
_RPT = _PROWS // _NC // _NS
_RPW = _PROWS // _NW
_G = 8
_NPAD = _N + 16
_WPT = 624
_WREM = _N - _WPT * _NS
_ZREM = _NPAD - _WPT * _NS
_DROWS = _NPAD * 4 // 128
_BLK = 1000

_mesh = plsc.VectorSubcoreMesh(core_axis_name="c", subcore_axis_name="s")
_sc_params = pltpu.CompilerParams(needs_layout_passes=False)


def _zeros16():
    return jnp.zeros((16,), jnp.float32)


_DWORDS = _NPAD * 4


@functools.partial(
    pl.kernel,
    out_type=jax.ShapeDtypeStruct((_NW * _DWORDS,), jnp.float32),
    mesh=_mesh,
    compiler_params=_sc_params,
    scratch_types=[
        pltpu.VMEM((16, 128), jnp.int32),
        pltpu.VMEM((_DWORDS,), jnp.float32),
    ],
)
def _deg_kernel(dst_ref, out_ref, dstbuf, deg_loc):
    cid = lax.axis_index("c")
    sid = lax.axis_index("s")
    wid = cid * _NS + sid

    def zero_body(i, _):
        deg_loc[pl.ds(i * 16, 16)] = _zeros16()
        return 0
    lax.fori_loop(0, _DWORDS // 16, zero_body, 0)

    ones = jnp.ones((16,), jnp.float32)
    base = _RPW * wid
    for r in range(_R):
        for c in range(_RPW // 16):
            pltpu.sync_copy(dst_ref.at[r, pl.ds(base + c * 16, 16)], dstbuf)

            def row_body(i, _):
                for j in range(8):
                    idx = dstbuf[i, pl.ds(j * 16, 16)] * 4 + r
                    plsc.addupdate_scatter(deg_loc, [idx], ones)
                return 0
            lax.fori_loop(0, 16, row_body, 0)

    pltpu.sync_copy(deg_loc, out_ref.at[pl.ds(wid * _DWORDS, _DWORDS)])


@functools.partial(
    pl.kernel,
    out_type=jax.ShapeDtypeStruct((_NC, _R, _N, _C), jnp.float32),
    mesh=_mesh,
    compiler_params=_sc_params,
    scratch_types=[
        pltpu.VMEM_SHARED((_NPAD, _C), jnp.float32),
        pltpu.VMEM((_G, 128), jnp.int32),
        pltpu.VMEM((_G, 128), jnp.int32),
        pltpu.VMEM((128, _C), jnp.float32),
        pltpu.VMEM((48, _C), jnp.float32),
        pltpu.SemaphoreType.DMA,
    ],
)
def _msg_kernel(ht_ref, src_ref, dst_ref, acc_ref,
                acc_sh, src_i, dst_i, rows, zrow, sem):
    cid = lax.axis_index("c")
    sid = lax.axis_index("s")

    def zrow_body(i, _):
        for j in range(_C // 16):
            zrow[i, pl.ds(j * 16, 16)] = _zeros16()
        return 0
    lax.fori_loop(0, 48, zrow_body, 0)

    base_row = (cid * _NS + sid) * _RPT
    nsup = _RPT // _G
    tail0 = _WPT * _NS

    for r in range(_R):
        def zero_copy(i, _):
            pltpu.sync_copy(zrow, acc_sh.at[pl.ds(sid * _WPT + i * 48, 48)])
            return 0
        lax.fori_loop(0, _WPT // 48, zero_copy, 0)

        @pl.when(sid < _ZREM // 8)
        def _zero_tail():
            pltpu.sync_copy(zrow.at[pl.ds(0, 8)],
                            acc_sh.at[pl.ds(tail0 + sid * 8, 8)])
        plsc.subcore_barrier()

        hview = ht_ref.at[r]

        def sup_body(g, _):
            row0 = base_row + g * _G
            pltpu.sync_copy(src_ref.at[r, pl.ds(row0, _G)], src_i)
            pltpu.sync_copy(dst_ref.at[r, pl.ds(row0, _G)], dst_i)
            for j in range(_G):
                pltpu.async_copy(hview.at[src_i.at[j]], rows, sem).wait()
                pltpu.sync_copy(rows, acc_sh.at[dst_i.at[j]], add=True)
            return 0
        lax.fori_loop(0, nsup, sup_body, 0)

        plsc.subcore_barrier()
        pltpu.sync_copy(
            acc_sh.at[pl.ds(sid * _WPT, _WPT)],
            acc_ref.at[cid, r, pl.ds(sid * _WPT, _WPT)],
        )

        @pl.when(sid < _WREM // 8)
        def _write_tail():
            pltpu.sync_copy(
                acc_sh.at[pl.ds(tail0 + sid * 8, 8)],
                acc_ref.at[cid, r, pl.ds(tail0 + sid * 8, 8)],
            )
        plsc.subcore_barrier()


def _dinv_body(degp_ref, dinv_ref):
    deg = jnp.sum(degp_ref[...], axis=0) + 1.0
    dinv_ref[...] = lax.rsqrt(deg)


def _dinv_call(degp):
    return pl.pallas_call(
        _dinv_body,
        grid=(_N // _BLK,),
        in_specs=[pl.BlockSpec((_NW, _BLK, 4), lambda i: (0, i, 0))],
        out_specs=pl.BlockSpec((_BLK, 4), lambda i: (i, 0)),
        out_shape=jax.ShapeDtypeStruct((_N, 4), jnp.float32),
    )(degp)


def _pre_body(h_ref, w_ref, dinv_ref, out_ref):
    h = h_ref[...]
    dinv = dinv_ref[...]
    for r in range(_R):
        hw = jnp.dot(h, w_ref[r], preferred_element_type=jnp.float32)
        out_ref[r] = hw * dinv[:, r:r + 1]


def _pre_call(h, w, dinv):
    return pl.pallas_call(
        _pre_body,
        grid=(_N // _BLK,),
        in_specs=[
            pl.BlockSpec((_BLK, _C), lambda i: (i, 0)),
            pl.BlockSpec((_R, _C, _C), lambda i: (0, 0, 0)),
            pl.BlockSpec((_BLK, 4), lambda i: (i, 0)),
        ],
        out_specs=pl.BlockSpec((_R, _BLK, _C), lambda i: (0, i, 0)),
        out_shape=jax.ShapeDtypeStruct((_R, _N, _C), jnp.float32),
    )(h, w, dinv)


def _post_body(acc_ref, ht_ref, dinv_ref, b_ref, w1_ref, w2_ref, out_ref,
               *, final_refs=None):
    dinv = dinv_ref[...]
    hs = []
    for r in range(_R):
        s = acc_ref[0, r] + acc_ref[1, r] + ht_ref[r]
        hr = jnp.maximum(s * dinv[:, r:r + 1] + b_ref[r:r + 1, :], 0.0)
        hs.append(hr)
    logits = []
    for r in range(_R):
        t = jnp.tanh(jnp.dot(hs[r], w1_ref[r], preferred_element_type=jnp.float32))
        a = jnp.sum(t * w2_ref[r:r + 1, :], axis=1, keepdims=True)
        logits.append(a)
    m = jnp.maximum(jnp.maximum(logits[0], logits[1]), logits[2])
    es = [jnp.exp(a - m) for a in logits]
    ssum = es[0] + es[1] + es[2]
    hout = (es[0] * hs[0] + es[1] * hs[1] + es[2] * hs[2]) / ssum
    if final_refs is None:
        out_ref[...] = hout
    else:
        pw_ref, pb_ref = final_refs
        out_ref[...] = (
            jnp.dot(hout, pw_ref[...], preferred_element_type=jnp.float32)
            + pb_ref[...]
        )


def _post1_body(acc_ref, ht_ref, dinv_ref, b_ref, w1_ref, w2_ref, out_ref):
    _post_body(acc_ref, ht_ref, dinv_ref, b_ref, w1_ref, w2_ref, out_ref)


def _post2_body(acc_ref, ht_ref, dinv_ref, b_ref, w1_ref, w2_ref,
                pw_ref, pb_ref, out_ref):
    _post_body(acc_ref, ht_ref, dinv_ref, b_ref, w1_ref, w2_ref, out_ref,
               final_refs=(pw_ref, pb_ref))


_POST_COMMON_SPECS = [
    pl.BlockSpec((_NC, _R, _BLK, _C), lambda i: (0, 0, i, 0)),
    pl.BlockSpec((_R, _BLK, _C), lambda i: (0, i, 0)),
    pl.BlockSpec((_BLK, 4), lambda i: (i, 0)),
    pl.BlockSpec((_R, _C), lambda i: (0, 0)),
    pl.BlockSpec((_R, _C, 64), lambda i: (0, 0, 0)),
    pl.BlockSpec((_R, 64), lambda i: (0, 0)),
]


def _post1_call(acc, ht, dinv, b, w1, w2):
    return pl.pallas_call(
        _post1_body,
        grid=(_N // _BLK,),
        in_specs=list(_POST_COMMON_SPECS),
        out_specs=pl.BlockSpec((_BLK, _C), lambda i: (i, 0)),
        out_shape=jax.ShapeDtypeStruct((_N, _C), jnp.float32),
    )(acc, ht, dinv, b, w1, w2)


def _post2_call(acc, ht, dinv, b, w1, w2, pw, pb):
    return pl.pallas_call(
        _post2_body,
        grid=(_N // _BLK,),
        in_specs=list(_POST_COMMON_SPECS) + [
            pl.BlockSpec((_C, 10), lambda i: (0, 0)),
            pl.BlockSpec((1, 10), lambda i: (0, 0)),
        ],
        out_specs=pl.BlockSpec((_BLK, 10), lambda i: (i, 0)),
        out_shape=jax.ShapeDtypeStruct((_N, 10), jnp.float32),
    )(acc, ht, dinv, b, w1, w2, pw, pb)


def kernel(x, edge_index_list, gcn_w, gcn_b, W1, W2, pred_w, pred_b):
    npad = _PROWS * 128 - _E
    src2d = jnp.concatenate(
        [edge_index_list[:, 0, :], jnp.zeros((_R, npad), jnp.int32)], axis=1
    ).reshape(_R, _PROWS, 128)
    dst2d = jnp.concatenate(
        [edge_index_list[:, 1, :], jnp.full((_R, npad), _N, jnp.int32)], axis=1
    ).reshape(_R, _PROWS, 128)

    degp = _deg_kernel(dst2d).reshape(_NW, _NPAD, 4)
    dinv = _dinv_call(degp)

    h = x
    for l in range(2):
        ht = _pre_call(h, gcn_w[l], dinv)
        acc = _msg_kernel(ht, src2d, dst2d)
        if l == 0:
            h = _post1_call(acc, ht, dinv, gcn_b[l], W1[l], W2[l][:, :, 0])
        else:
            h = _post2_call(acc, ht, dinv, gcn_b[l], W1[l], W2[l][:, :, 0],
                            pred_w, pred_b.reshape(1, 10))
    return h
